# Initial kernel scaffold; baseline (speedup 1.0000x reference)
import jax
import jax.numpy as jnp
from jax.experimental import pallas as pl


def kernel(inputs, codebooks, random_vector):
    raise NotImplementedError("write your pallas kernel here")


# speedup vs baseline: 2.4876x; 2.4876x over previous
import jax
import jax.numpy as jnp
from jax.experimental import pallas as pl
from jax.experimental.pallas import tpu as pltpu

NE = 1024
ED = 64
NB = 16
TPB = 1024
NTOK = NB * TPB
EPS = 1e-12


def _nsvq_body(x_ref, c_ref, rv_ref, out_ref, plex_ref, counts_ref):
    b = pl.program_id(0)
    x_t = x_ref[0]
    cb = c_ref[...]
    rv = rv_ref[...]

    scores = jax.lax.dot(cb, x_t, preferred_element_type=jnp.float32)
    cnorm = jnp.sum(cb * cb, axis=1, keepdims=True)
    neg = cnorm - 2.0 * scores

    md = jnp.min(neg, axis=0, keepdims=True)
    idx = jnp.argmin(neg, axis=0).reshape(1, TPB)

    xsq = jnp.sum(x_t * x_t, axis=0, keepdims=True)
    dist = jnp.maximum(xsq + md, 0.0)

    rv_t = rv.T
    rnorm = jnp.sqrt(jnp.sum(rv_t * rv_t, axis=0, keepdims=True))
    scale = jnp.sqrt(dist) / rnorm + EPS
    out_ref[0] = x_t + rv_t * scale

    code_iota = jax.lax.broadcasted_iota(jnp.int32, (NE, TPB), 0)
    onehot = (code_iota == idx).astype(jnp.float32)
    cnt = jnp.sum(onehot, axis=1, keepdims=True)

    @pl.when(b == 0)
    def _init():
        counts_ref[...] = jnp.zeros_like(counts_ref)

    counts_ref[...] += cnt

    @pl.when(b == NB - 1)
    def _finish():
        p = counts_ref[...] / NTOK
        plex_ref[...] = jnp.exp(-jnp.sum(p * jnp.log(p + 1e-10))).reshape(1, 1)


def kernel(inputs, codebooks, random_vector):
    x = inputs.reshape(NB, ED, TPB)
    out, plex = pl.pallas_call(
        _nsvq_body,
        grid=(NB,),
        in_specs=[
            pl.BlockSpec((1, ED, TPB), lambda b: (b, 0, 0)),
            pl.BlockSpec((NE, ED), lambda b: (0, 0)),
            pl.BlockSpec((TPB, ED), lambda b: (b, 0)),
        ],
        out_specs=[
            pl.BlockSpec((1, ED, TPB), lambda b: (b, 0, 0)),
            pl.BlockSpec((1, 1), lambda b: (0, 0)),
        ],
        out_shape=[
            jax.ShapeDtypeStruct((NB, ED, TPB), jnp.float32),
            jax.ShapeDtypeStruct((1, 1), jnp.float32),
        ],
        scratch_shapes=[pltpu.VMEM((NE, 1), jnp.float32)],
        compiler_params=pltpu.CompilerParams(
            dimension_semantics=("arbitrary",),
        ),
    )(x, codebooks, random_vector)
    return out.reshape(NB, ED, 32, 32), plex.reshape(())


# speedup vs baseline: 2.6219x; 1.0540x over previous
import jax
import jax.numpy as jnp
from jax.experimental import pallas as pl
from jax.experimental.pallas import tpu as pltpu

NE = 1024
ED = 64
NB = 16
TPB = 1024
NTOK = NB * TPB
EPS = 1e-12


def _nsvq_body(x_ref, c_ref, rv_ref, out_ref, plex_ref, counts_ref):
    b = pl.program_id(0)
    x_t = x_ref[0]
    cb = c_ref[...]
    rv = rv_ref[...]

    cnorm = jnp.sum(cb * cb, axis=1, keepdims=True)
    cb_aug = jnp.concatenate([cb, cnorm], axis=1)
    x_aug = jnp.concatenate(
        [-2.0 * x_t, jnp.ones((1, TPB), jnp.float32)], axis=0)
    neg = jax.lax.dot(cb_aug, x_aug, preferred_element_type=jnp.float32)

    md = jnp.min(neg, axis=0, keepdims=True)

    xsq = jnp.sum(x_t * x_t, axis=0, keepdims=True)
    dist = jnp.maximum(xsq + md, 0.0)

    rv_t = rv.T
    rnorm = jnp.sqrt(jnp.sum(rv_t * rv_t, axis=0, keepdims=True))
    scale = jnp.sqrt(dist) / rnorm + EPS
    out_ref[0] = x_t + rv_t * scale

    cnt = jnp.sum(jnp.where(neg == md, 1.0, 0.0), axis=1, keepdims=True)

    @pl.when(b == 0)
    def _init():
        counts_ref[...] = jnp.zeros_like(counts_ref)

    counts_ref[...] += cnt

    @pl.when(b == NB - 1)
    def _finish():
        p = counts_ref[...] / NTOK
        plex_ref[...] = jnp.exp(-jnp.sum(p * jnp.log(p + 1e-10))).reshape(1, 1)


def kernel(inputs, codebooks, random_vector):
    x = inputs.reshape(NB, ED, TPB)
    out, plex = pl.pallas_call(
        _nsvq_body,
        grid=(NB,),
        in_specs=[
            pl.BlockSpec((1, ED, TPB), lambda b: (b, 0, 0)),
            pl.BlockSpec((NE, ED), lambda b: (0, 0)),
            pl.BlockSpec((TPB, ED), lambda b: (b, 0)),
        ],
        out_specs=[
            pl.BlockSpec((1, ED, TPB), lambda b: (b, 0, 0)),
            pl.BlockSpec((1, 1), lambda b: (0, 0)),
        ],
        out_shape=[
            jax.ShapeDtypeStruct((NB, ED, TPB), jnp.float32),
            jax.ShapeDtypeStruct((1, 1), jnp.float32),
        ],
        scratch_shapes=[pltpu.VMEM((NE, 1), jnp.float32)],
        compiler_params=pltpu.CompilerParams(
            dimension_semantics=("arbitrary",),
        ),
    )(x, codebooks, random_vector)
    return out.reshape(NB, ED, 32, 32), plex.reshape(())


# speedup vs baseline: 2.7128x; 1.0347x over previous
import jax
import jax.numpy as jnp
from jax.experimental import pallas as pl
from jax.experimental.pallas import tpu as pltpu

NE = 1024
ED = 64
NB = 16
TPB = 1024
NTOK = NB * TPB
EPS = 1e-12

NB_PER_STEP = 2
GRID = NB // NB_PER_STEP
TPS = NB_PER_STEP * TPB


def _nsvq_body(x_ref, c_ref, rv_ref, out_ref, plex_ref, counts_ref):
    g = pl.program_id(0)
    x_t = jnp.concatenate(
        [x_ref[i] for i in range(NB_PER_STEP)], axis=1)
    cb = c_ref[...]
    rv = rv_ref[...]

    cnorm = jnp.sum(cb * cb, axis=1, keepdims=True)
    cb_aug = jnp.concatenate([cb, cnorm], axis=1)
    x_aug = jnp.concatenate(
        [-2.0 * x_t, jnp.ones((1, TPS), jnp.float32)], axis=0)
    neg = jax.lax.dot(cb_aug, x_aug, preferred_element_type=jnp.float32)

    md = jnp.min(neg, axis=0, keepdims=True)

    xsq = jnp.sum(x_t * x_t, axis=0, keepdims=True)
    dist = jnp.maximum(xsq + md, 0.0)

    rv_t = rv.T
    rnorm = jnp.sqrt(jnp.sum(rv_t * rv_t, axis=0, keepdims=True))
    scale = jnp.sqrt(dist) / rnorm + EPS
    out = x_t + rv_t * scale
    for i in range(NB_PER_STEP):
        out_ref[i] = out[:, i * TPB:(i + 1) * TPB]

    cnt = jnp.sum(jnp.where(neg == md, 1.0, 0.0), axis=1, keepdims=True)

    @pl.when(g == 0)
    def _init():
        counts_ref[...] = jnp.zeros_like(counts_ref)

    counts_ref[...] += cnt

    @pl.when(g == GRID - 1)
    def _finish():
        p = counts_ref[...] / NTOK
        plex_ref[...] = jnp.exp(-jnp.sum(p * jnp.log(p + 1e-10))).reshape(1, 1)


def kernel(inputs, codebooks, random_vector):
    x = inputs.reshape(NB, ED, TPB)
    out, plex = pl.pallas_call(
        _nsvq_body,
        grid=(GRID,),
        in_specs=[
            pl.BlockSpec((NB_PER_STEP, ED, TPB), lambda g: (g, 0, 0)),
            pl.BlockSpec((NE, ED), lambda g: (0, 0)),
            pl.BlockSpec((TPS, ED), lambda g: (g, 0)),
        ],
        out_specs=[
            pl.BlockSpec((NB_PER_STEP, ED, TPB), lambda g: (g, 0, 0)),
            pl.BlockSpec((1, 1), lambda g: (0, 0)),
        ],
        out_shape=[
            jax.ShapeDtypeStruct((NB, ED, TPB), jnp.float32),
            jax.ShapeDtypeStruct((1, 1), jnp.float32),
        ],
        scratch_shapes=[pltpu.VMEM((NE, 1), jnp.float32)],
        compiler_params=pltpu.CompilerParams(
            dimension_semantics=("arbitrary",),
        ),
    )(x, codebooks, random_vector)
    return out.reshape(NB, ED, 32, 32), plex.reshape(())
